# baseline (speedup 1.0000x reference)
import functools

import jax
import jax.numpy as jnp
from jax import lax
from jax.experimental import pallas as pl
from jax.experimental.pallas import tpu as pltpu
from jax.experimental.pallas import tpu_sc as plsc

F32 = jnp.float32
BF16 = jnp.bfloat16

NC, NS = 2, 16
NW = NC * NS
DN = 16
G = 8


def _sc_mesh():
    return plsc.VectorSubcoreMesh(core_axis_name="c", subcore_axis_name="s",
                                  num_cores=NC, num_subcores=NS)


def _gather_rows(table, idx, chunk=2000):
    M = idx.shape[0]
    b_w = M // NW
    n_chunks = b_w // chunk

    @functools.partial(
        pl.kernel,
        out_type=jax.ShapeDtypeStruct((M, DN), F32),
        mesh=_sc_mesh(),
        scratch_types=[pltpu.VMEM((chunk,), jnp.int32),
                       pltpu.VMEM((chunk, DN), F32),
                       pltpu.SemaphoreType.DMA],
        compiler_params=pltpu.CompilerParams(use_tc_tiling_on_sc=False),
    )
    def k(table_hbm, idx_hbm, out_hbm, idx_v, rows_v, sem):
        wid = lax.axis_index("c") * NS + lax.axis_index("s")
        base = wid * b_w

        def body(i, carry):
            off = base + i * chunk
            pltpu.sync_copy(idx_hbm.at[pl.ds(off, chunk)], idx_v)
            pltpu.async_copy(table_hbm.at[idx_v], rows_v, sem).wait()
            pltpu.sync_copy(rows_v, out_hbm.at[pl.ds(off, chunk)])
            return carry

        lax.fori_loop(0, n_chunks, body, 0)

    return k(table, idx)


def _segment_sum(e, dst, n_pad, chunk=1000):
    M = e.shape[0]
    b_w = M // NW
    n_chunks = b_w // chunk
    rpt = n_pad // NS
    zeros = jnp.zeros((rpt, DN), F32)

    @functools.partial(
        pl.kernel,
        out_type=jax.ShapeDtypeStruct((NC * n_pad, DN), F32),
        mesh=_sc_mesh(),
        scratch_types=[pltpu.VMEM((chunk,), jnp.int32),
                       pltpu.VMEM((chunk, DN), F32),
                       pltpu.VMEM_SHARED((n_pad, DN), F32),
                       pltpu.SemaphoreType.DMA],
        compiler_params=pltpu.CompilerParams(use_tc_tiling_on_sc=False),
    )
    def k(e_hbm, dst_hbm, z_hbm, out_hbm, idx_v, rows_v, acc_sh, sem):
        c = lax.axis_index("c")
        s = lax.axis_index("s")
        pltpu.sync_copy(z_hbm, acc_sh.at[pl.ds(s * rpt, rpt)])
        plsc.subcore_barrier()
        base = (c * NS + s) * b_w

        def body(i, carry):
            off = base + i * chunk
            pltpu.sync_copy(dst_hbm.at[pl.ds(off, chunk)], idx_v)
            pltpu.sync_copy(e_hbm.at[pl.ds(off, chunk)], rows_v)
            pltpu.sync_copy(rows_v, acc_sh.at[idx_v], add=True)
            return carry

        lax.fori_loop(0, n_chunks, body, 0)
        plsc.subcore_barrier()
        pltpu.sync_copy(acc_sh.at[pl.ds(s * rpt, rpt)],
                        out_hbm.at[pl.ds(c * n_pad + s * rpt, rpt)])

    return k(e, dst, zeros)


BRP = 2000


def _edge_mlp(mp, pieces, b1, hidden, out_wt, out_b, final=None,
              narrow=False):
    grid = mp // BRP
    in_specs = []
    operands = []
    for arr, off, _ in pieces:
        in_specs.append(pl.BlockSpec((BRP, 128), lambda i, o=off: (i + o, 0)))
        operands.append(arr)

    def whole(a):
        in_specs.append(pl.BlockSpec(a.shape, lambda i: (0, 0)))
        operands.append(a)

    for _, _, w in pieces:
        whole(w)
    whole(b1)
    for w, b in hidden:
        whole(w)
        whole(b)
    whole(out_wt)
    whole(out_b)

    n_p = len(pieces)
    n_h = len(hidden)

    def body(*refs):
        xs = refs[:n_p]
        wps = refs[n_p:2 * n_p]
        k = 2 * n_p
        b1r = refs[k]
        k += 1
        hs = []
        for _ in range(n_h):
            hs.append((refs[k], refs[k + 1]))
            k += 2
        owr, obr = refs[k], refs[k + 1]
        out_refs = refs[k + 2:]
        wvs = [wr[...] for wr in wps]
        b1v = b1r[...]
        hvs = [(w[...], b[...]) for w, b in hs]
        owv, obv = owr[...], obr[...]
        HB = BRP // 2
        halves = []
        for u in range(2):
            acc = None
            for xr, wv in zip(xs, wvs):
                xv = xr[pl.ds(u * HB, HB), :]
                t = jnp.dot(xv.astype(BF16), wv,
                            preferred_element_type=F32)
                acc = t if acc is None else acc + t
            h = jnp.maximum(acc + b1v, 0.0)
            for wv2, bv2 in hvs:
                h = jnp.maximum(
                    jnp.dot(h.astype(BF16), wv2,
                            preferred_element_type=F32) + bv2, 0.0)
            o = jnp.dot(h.astype(BF16), owv,
                        preferred_element_type=F32) + obv
            if final == "sigmoid":
                o = jax.nn.sigmoid(o)
            halves.append(o)
        o = jnp.concatenate(halves, axis=0)
        out_refs[0][...] = o
        if narrow:
            out_refs[1][...] = jnp.concatenate(
                [o[:, DN * j:DN * j + 1] for j in range(G)], axis=1)

    out_specs = [pl.BlockSpec((BRP, 128), lambda i: (i, 0))]
    out_shape = [jax.ShapeDtypeStruct((mp, 128), F32)]
    if narrow:
        out_specs.append(pl.BlockSpec((BRP, G), lambda i: (i, 0)))
        out_shape.append(jax.ShapeDtypeStruct((mp, G), F32))

    res = pl.pallas_call(
        body,
        grid=(grid,),
        in_specs=in_specs,
        out_specs=out_specs,
        out_shape=out_shape,
    )(*operands)
    return res if narrow else res[0]


def _fused_mid(mp, gp0, gp1, e1, eap, sets):
    grid = mp // BRP
    OB = mp // BRP

    arrays = [gp1, gp1, e1, eap, gp0, gp0]
    offs = [0, OB, 0, 0, 0, OB]
    in_specs = [pl.BlockSpec((BRP, 128), lambda i, o=o: (i + o, 0))
                for o in offs]
    operands = list(arrays)

    flat_w = []

    def whole(a):
        in_specs.append(pl.BlockSpec(a.shape, lambda i: (0, 0)))
        operands.append(a)
        flat_w.append(a)

    layout = []
    for name in ("w2", "wh", "c1"):
        st = sets[name]
        start = len(flat_w)
        for wp in st["wps"]:
            whole(wp)
        whole(st["b1"])
        for w, b in st["hidden"]:
            whole(w)
            whole(b)
        whole(st["out"][0])
        whole(st["out"][1])
        layout.append((start, len(st["wps"]), len(st["hidden"])))

    def chain(refs, xbfs, lay, final=None):
        start, npc, nh = lay
        k = 6 + start
        acc = None
        for j in range(npc):
            t = jnp.dot(xbfs[j], refs[k + j][...],
                        preferred_element_type=F32)
            acc = t if acc is None else acc + t
        k += npc
        h = jnp.maximum(acc + refs[k][...], 0.0)
        k += 1
        for _ in range(nh):
            h = jnp.maximum(
                jnp.dot(h.astype(BF16), refs[k][...],
                        preferred_element_type=F32) + refs[k + 1][...], 0.0)
            k += 2
        o = jnp.dot(h.astype(BF16), refs[k][...],
                    preferred_element_type=F32) + refs[k + 1][...]
        if final == "sigmoid":
            o = jax.nn.sigmoid(o)
        return o

    def body(*refs):
        out_ec1, out_ewn = refs[-2], refs[-1]
        HB = BRP // 2
        ec1_h = []
        ew_h = []
        for u in range(2):
            sl = pl.ds(u * HB, HB)
            g1a = refs[0][sl, :].astype(BF16)
            g1b = refs[1][sl, :].astype(BF16)
            e1v = refs[2][sl, :].astype(BF16)
            eav = refs[3][sl, :].astype(BF16)
            g0a = refs[4][sl, :].astype(BF16)
            g0b = refs[5][sl, :].astype(BF16)
            e2 = chain(refs, [g1a, g1b, e1v], layout[0]).astype(BF16)
            ew = chain(refs, [eav, e1v, e2], layout[1], final="sigmoid")
            ec1 = chain(refs, [g0a, g0b, ew.astype(BF16), eav, e1v, e2],
                        layout[2])
            ec1_h.append(ec1)
            ew_h.append(ew)
        out_ec1[...] = jnp.concatenate(ec1_h, axis=0)
        ew = jnp.concatenate(ew_h, axis=0)
        out_ewn[...] = jnp.concatenate(
            [ew[:, DN * j:DN * j + 1] for j in range(G)], axis=1)

    return pl.pallas_call(
        body,
        grid=(grid,),
        in_specs=in_specs,
        out_specs=[pl.BlockSpec((BRP, 128), lambda i: (i, 0)),
                   pl.BlockSpec((BRP, G), lambda i: (i, 0))],
        out_shape=[jax.ShapeDtypeStruct((mp, 128), F32),
                   jax.ShapeDtypeStruct((mp, G), F32)],
    )(*operands)


def _node_mlp(n_out, br, pieces, b1, hidden, out_wt, out_b, final=None):
    grid = n_out // br
    in_specs = []
    operands = []
    for arr, blk_off, _ in pieces:
        in_specs.append(pl.BlockSpec((br, arr.shape[1]),
                                     lambda i, o=blk_off: (i + o, 0)))
        operands.append(arr)

    def whole(a):
        in_specs.append(pl.BlockSpec(a.shape, lambda i: (0, 0)))
        operands.append(a)

    for _, _, wp in pieces:
        whole(wp)
    whole(b1)
    for w, b in hidden:
        whole(w)
        whole(b)
    whole(out_wt)
    whole(out_b)

    n_p = len(pieces)
    n_h = len(hidden)
    out_width = out_wt.shape[1]

    def body(*refs):
        xs = refs[:n_p]
        wps = refs[n_p:2 * n_p]
        k = 2 * n_p
        b1r = refs[k]
        k += 1
        hs = []
        for _ in range(n_h):
            hs.append((refs[k], refs[k + 1]))
            k += 2
        owr, obr = refs[k], refs[k + 1]
        out_ref = refs[k + 2]
        acc = None
        for xr, wr in zip(xs, wps):
            t = jnp.dot(xr[...].astype(BF16), wr[...],
                        preferred_element_type=F32)
            acc = t if acc is None else acc + t
        h = jnp.maximum(acc + b1r[...], 0.0)
        for wr2, br2 in hs:
            h = jnp.maximum(
                jnp.dot(h.astype(BF16), wr2[...],
                        preferred_element_type=F32) + br2[...], 0.0)
        o = jnp.dot(h.astype(BF16), owr[...],
                    preferred_element_type=F32) + obr[...]
        if final == "sigmoid":
            o = jax.nn.sigmoid(o)
        out_ref[...] = o

    return pl.pallas_call(
        body,
        grid=(grid,),
        in_specs=in_specs,
        out_specs=pl.BlockSpec((br, out_width), lambda i: (i, 0)),
        out_shape=jax.ShapeDtypeStruct((n_out, out_width), F32),
    )(*operands)


def _repack_ea(ea, n_edges):
    mp = n_edges // G
    grid = mp // BRP
    gblk = mp // BRP

    def body(*refs):
        out_ref = refs[G]
        z = jnp.zeros((BRP, DN - 4), F32)
        out_ref[...] = jnp.concatenate(
            [jnp.concatenate([refs[g][...], z], axis=1) for g in range(G)],
            axis=1)

    return pl.pallas_call(
        body,
        grid=(grid,),
        in_specs=[pl.BlockSpec((BRP, 4), lambda i, g=g: (i + g * gblk, 0))
                  for g in range(G)],
        out_specs=pl.BlockSpec((BRP, 128), lambda i: (i, 0)),
        out_shape=jax.ShapeDtypeStruct((mp, 128), F32),
    )(*([ea] * G))


def _pad_rows(wt, rows):
    return jnp.pad(wt, ((0, rows - wt.shape[0]), (0, 0)))


def _pad_cols(wt, cols):
    return jnp.pad(wt, ((0, 0), (0, cols - wt.shape[1])))


def _bd(w):
    return jnp.kron(jnp.eye(G, dtype=w.dtype), w)


def kernel(x, edge_index, edge_attr, params):
    n_nodes = x.shape[0]
    n_edges = edge_index.shape[1]
    n_pad = ((n_nodes + NS * 8 - 1) // (NS * 8)) * (NS * 8)
    me = n_edges // G
    mg = 2 * n_edges // G
    OB = (n_edges // G) // BRP

    dst = edge_index[1].astype(jnp.int32)
    src = edge_index[0].astype(jnp.int32)
    idx_cat = jnp.concatenate([dst, src])
    idx_p = jnp.transpose(idx_cat.reshape(G, mg)).reshape(2 * n_edges)
    dst_p = jnp.transpose(dst.reshape(G, n_edges // G)).reshape(n_edges)

    xp = jnp.zeros((n_pad, DN), F32).at[:n_nodes, :3].set(x)
    eap = _repack_ea(edge_attr.astype(F32), n_edges)

    BR_N = n_pad // 16

    def bf(w):
        return w.astype(BF16)

    def rel_first_gather_w(W1t, h):
        Wd = _pad_rows(W1t[0:3], DN)
        Ws = _pad_rows(W1t[3:6], DN)
        WA = jnp.zeros((128, G * h), F32)
        WB = jnp.zeros((128, G * h), F32)
        for a in range(4):
            WA = WA.at[DN * a:DN * a + DN, 2 * a * h:(2 * a + 1) * h].set(Wd)
            WA = WA.at[64 + DN * a:64 + DN * a + DN,
                       2 * a * h:(2 * a + 1) * h].set(Ws)
            WB = WB.at[DN * a:DN * a + DN,
                       (2 * a + 1) * h:(2 * a + 2) * h].set(Wd)
            WB = WB.at[64 + DN * a:64 + DN * a + DN,
                       (2 * a + 1) * h:(2 * a + 2) * h].set(Ws)
        return WA, WB

    def rel_call(p, e_pieces, gpk):
        (W1, b1), (W2, b2), (W3, b3), (W4, b4) = p
        h = W1.shape[0]
        W1t = W1.T
        WA, WB = rel_first_gather_w(W1t, h)
        pieces = [(gpk, 0, bf(WA)), (gpk, OB, bf(WB))]
        col = 6
        for arr, d in e_pieces:
            wp = _pad_rows(W1t[col:col + d], DN)
            col += d
            pieces.append((arr, 0, bf(_bd(wp))))
        hidden = [(bf(_bd(W2.T)), jnp.tile(b2[None, :], (1, G))),
                  (bf(_bd(W3.T)), jnp.tile(b3[None, :], (1, G)))]
        owt = bf(_bd(_pad_cols(W4.T, DN)))
        ob = jnp.tile(_pad_cols(b4[None, :], DN), (1, G))
        return _edge_mlp(me, pieces, jnp.tile(b1[None, :], (1, G)),
                         hidden, owt, ob)

    def whead_call(p, e_pieces):
        (W1, b1), (W2, b2), (W3, b3), (W4, b4) = p
        h = W1.shape[0]
        W1t = W1.T
        pieces = []
        col = 0
        for arr, d in e_pieces:
            wp = _pad_rows(W1t[col:col + d], DN)
            col += d
            pieces.append((arr, 0, bf(_bd(wp))))
        hidden = [(bf(_bd(W2.T)), jnp.tile(b2[None, :], (1, G))),
                  (bf(_bd(W3.T)), jnp.tile(b3[None, :], (1, G)))]
        owt = bf(_bd(_pad_cols(W4.T, DN)))
        ob = jnp.tile(_pad_cols(b4[None, :], DN), (1, G))
        return _edge_mlp(me, pieces, jnp.tile(b1[None, :], (1, G)),
                         hidden, owt, ob, final="sigmoid", narrow=True)

    def obj_call(p, xtab, agg):
        (W1, b1), (W2, b2), (W3, b3), (W4, b4) = p
        W1t = W1.T
        eo = W1t.shape[0] - 3
        wx = bf(_pad_rows(W1t[0:3], DN))
        wa = bf(_pad_rows(W1t[3:3 + eo], DN))
        pieces = [(xtab, 0, wx), (agg, 0, wa), (agg, n_pad // BR_N, wa)]
        hidden = [(bf(W2.T), b2[None, :]), (bf(W3.T), b3[None, :])]
        return _node_mlp(n_pad, BR_N, pieces, b1[None, :], hidden,
                         bf(_pad_cols(W4.T, DN)), _pad_cols(b4[None, :], DN))

    def nhead_call(p, tabs, final):
        (W1, b1), (W2, b2), (W3, b3), (W4, b4) = p
        W1t = W1.T
        pieces = [(tab, 0, bf(_pad_rows(W1t[3 * i:3 * i + 3], DN)))
                  for i, tab in enumerate(tabs)]
        hidden = [(bf(W2.T), b2[None, :]), (bf(W3.T), b3[None, :])]
        return _node_mlp(n_pad, BR_N, pieces, b1[None, :], hidden,
                         bf(W4.T), b4[None, :], final=final)

    g0 = _gather_rows(xp, idx_p)
    gp0 = g0.reshape(mg, 128)
    e1 = rel_call(params['in_w1']['rel'], [(eap, 4)], gp0)
    a1 = _segment_sum(e1.reshape(n_edges, DN), dst_p, n_pad)
    x1 = obj_call(params['in_w1']['obj'], xp, a1)

    g1 = _gather_rows(x1, idx_p)
    gp1 = g1.reshape(mg, 128)

    def rel_wset(p, e_dims, with_gather=True):
        (W1, b1), (W2, b2), (W3, b3), (W4, b4) = p
        h = W1.shape[0]
        W1t = W1.T
        wps = []
        col = 0
        if with_gather:
            WA, WB = rel_first_gather_w(W1t, h)
            wps += [bf(WA), bf(WB)]
            col = 6
        for d in e_dims:
            wps.append(bf(_bd(_pad_rows(W1t[col:col + d], DN))))
            col += d
        return {
            "wps": wps,
            "b1": jnp.tile(b1[None, :], (1, G)),
            "hidden": [(bf(_bd(W2.T)), jnp.tile(b2[None, :], (1, G))),
                       (bf(_bd(W3.T)), jnp.tile(b3[None, :], (1, G)))],
            "out": (bf(_bd(_pad_cols(W4.T, DN))),
                    jnp.tile(_pad_cols(b4[None, :], DN), (1, G))),
        }

    sets = {
        "w2": rel_wset(params['in_w2']['rel'], [4]),
        "wh": rel_wset(params['W'], [4, 4, 4], with_gather=False),
        "c1": rel_wset(params['in_c1']['rel'], [1, 4, 4, 4]),
    }
    ec1, ewn = _fused_mid(me, gp0, gp1, e1, eap, sets)
    ac1 = _segment_sum(ec1.reshape(n_edges, DN), dst_p, n_pad)
    xc1 = obj_call(params['in_c1']['obj'], xp, ac1)

    gc2 = _gather_rows(xc1, idx_p)
    gpc2 = gc2.reshape(mg, 128)
    ec2 = rel_call(params['in_c2']['rel'], [(ec1, 8)], gpc2)
    ac2 = _segment_sum(ec2.reshape(n_edges, DN), dst_p, n_pad)
    xc2 = obj_call(params['in_c2']['obj'], xc1, ac2)

    gc3 = _gather_rows(xc2, idx_p)
    gpc3 = gc3.reshape(mg, 128)
    ec3 = rel_call(params['in_c3']['rel'], [(ec2, 8)], gpc3)
    ac3 = _segment_sum(ec3.reshape(n_edges, DN), dst_p, n_pad)
    xc3 = obj_call(params['in_c3']['obj'], xc2, ac3)

    tabs = [xp, xc1, xc2, xc3]
    beta = nhead_call(params['B'], tabs, "sigmoid")
    xc = nhead_call(params['X'], tabs, None)

    ew = jnp.transpose(ewn).reshape(n_edges, 1)
    return ew, xc[:n_nodes], beta[:n_nodes]


# speedup vs baseline: 1.0682x; 1.0682x over previous
import functools

import jax
import jax.numpy as jnp
from jax import lax
from jax.experimental import pallas as pl
from jax.experimental.pallas import tpu as pltpu
from jax.experimental.pallas import tpu_sc as plsc

F32 = jnp.float32
BF16 = jnp.bfloat16

NC, NS = 2, 16
NW = NC * NS
DN = 16
G = 8


def _sc_mesh():
    return plsc.VectorSubcoreMesh(core_axis_name="c", subcore_axis_name="s",
                                  num_cores=NC, num_subcores=NS)


def _gather_rows(table, idx, chunk=5000):
    M = idx.shape[0]
    b_w = M // NW
    n_chunks = b_w // chunk

    @functools.partial(
        pl.kernel,
        out_type=jax.ShapeDtypeStruct((M, DN), F32),
        mesh=_sc_mesh(),
        scratch_types=[pltpu.VMEM((chunk,), jnp.int32),
                       pltpu.VMEM((chunk, DN), F32),
                       pltpu.SemaphoreType.DMA],
        compiler_params=pltpu.CompilerParams(use_tc_tiling_on_sc=False),
    )
    def k(table_hbm, idx_hbm, out_hbm, idx_v, rows_v, sem):
        wid = lax.axis_index("c") * NS + lax.axis_index("s")
        base = wid * b_w

        def body(i, carry):
            off = base + i * chunk
            pltpu.sync_copy(idx_hbm.at[pl.ds(off, chunk)], idx_v)
            pltpu.async_copy(table_hbm.at[idx_v], rows_v, sem).wait()
            pltpu.sync_copy(rows_v, out_hbm.at[pl.ds(off, chunk)])
            return carry

        lax.fori_loop(0, n_chunks, body, 0)

    return k(table, idx)


def _segment_sum(e, dst, n_pad, chunk=1000):
    M = e.shape[0]
    b_w = M // NW
    n_chunks = b_w // chunk
    rpt = n_pad // NS
    zeros = jnp.zeros((rpt, DN), F32)

    @functools.partial(
        pl.kernel,
        out_type=jax.ShapeDtypeStruct((NC * n_pad, DN), F32),
        mesh=_sc_mesh(),
        scratch_types=[pltpu.VMEM((chunk,), jnp.int32),
                       pltpu.VMEM((chunk, DN), F32),
                       pltpu.VMEM_SHARED((n_pad, DN), F32),
                       pltpu.SemaphoreType.DMA],
        compiler_params=pltpu.CompilerParams(use_tc_tiling_on_sc=False),
    )
    def k(e_hbm, dst_hbm, z_hbm, out_hbm, idx_v, rows_v, acc_sh, sem):
        c = lax.axis_index("c")
        s = lax.axis_index("s")
        pltpu.sync_copy(z_hbm, acc_sh.at[pl.ds(s * rpt, rpt)])
        plsc.subcore_barrier()
        base = (c * NS + s) * b_w

        def body(i, carry):
            off = base + i * chunk
            pltpu.sync_copy(dst_hbm.at[pl.ds(off, chunk)], idx_v)
            pltpu.sync_copy(e_hbm.at[pl.ds(off, chunk)], rows_v)
            pltpu.sync_copy(rows_v, acc_sh.at[idx_v], add=True)
            return carry

        lax.fori_loop(0, n_chunks, body, 0)
        plsc.subcore_barrier()
        pltpu.sync_copy(acc_sh.at[pl.ds(s * rpt, rpt)],
                        out_hbm.at[pl.ds(c * n_pad + s * rpt, rpt)])

    return k(e, dst, zeros)


BRP = 2000


def _edge_mlp(mp, pieces, b1, hidden, out_wt, out_b, final=None,
              narrow=False):
    grid = mp // BRP
    in_specs = []
    operands = []
    for arr, off, _ in pieces:
        in_specs.append(pl.BlockSpec((BRP, 128), lambda i, o=off: (i + o, 0)))
        operands.append(arr)

    def whole(a):
        in_specs.append(pl.BlockSpec(a.shape, lambda i: (0, 0)))
        operands.append(a)

    for _, _, w in pieces:
        whole(w)
    whole(b1)
    for w, b in hidden:
        whole(w)
        whole(b)
    whole(out_wt)
    whole(out_b)

    n_p = len(pieces)
    n_h = len(hidden)

    def body(*refs):
        xs = refs[:n_p]
        wps = refs[n_p:2 * n_p]
        k = 2 * n_p
        b1r = refs[k]
        k += 1
        hs = []
        for _ in range(n_h):
            hs.append((refs[k], refs[k + 1]))
            k += 2
        owr, obr = refs[k], refs[k + 1]
        out_refs = refs[k + 2:]
        acc = None
        for xr, wr in zip(xs, wps):
            t = jnp.dot(xr[...].astype(BF16), wr[...],
                        preferred_element_type=F32)
            acc = t if acc is None else acc + t
        h = jnp.maximum(acc + b1r[...], 0.0)
        for wr2, br2 in hs:
            h = jnp.maximum(
                jnp.dot(h.astype(BF16), wr2[...],
                        preferred_element_type=F32) + br2[...], 0.0)
        o = jnp.dot(h.astype(BF16), owr[...],
                    preferred_element_type=F32) + obr[...]
        if final == "sigmoid":
            o = jax.nn.sigmoid(o)
        out_refs[0][...] = o
        if narrow:
            out_refs[1][...] = jnp.concatenate(
                [o[:, DN * j:DN * j + 1] for j in range(G)], axis=1)

    out_specs = [pl.BlockSpec((BRP, 128), lambda i: (i, 0))]
    out_shape = [jax.ShapeDtypeStruct((mp, 128), F32)]
    if narrow:
        out_specs.append(pl.BlockSpec((BRP, G), lambda i: (i, 0)))
        out_shape.append(jax.ShapeDtypeStruct((mp, G), F32))

    res = pl.pallas_call(
        body,
        grid=(grid,),
        in_specs=in_specs,
        out_specs=out_specs,
        out_shape=out_shape,
    )(*operands)
    return res if narrow else res[0]


def _fused_mid(mp, gp0, gp1, e1, eap, sets):
    grid = mp // BRP
    OB = mp // BRP

    arrays = [gp1, gp1, e1, eap, gp0, gp0]
    offs = [0, OB, 0, 0, 0, OB]
    in_specs = [pl.BlockSpec((BRP, 128), lambda i, o=o: (i + o, 0))
                for o in offs]
    operands = list(arrays)

    flat_w = []

    def whole(a):
        in_specs.append(pl.BlockSpec(a.shape, lambda i: (0, 0)))
        operands.append(a)
        flat_w.append(a)

    layout = []
    for name in ("w2", "wh", "c1"):
        st = sets[name]
        start = len(flat_w)
        for wp in st["wps"]:
            whole(wp)
        whole(st["b1"])
        for w, b in st["hidden"]:
            whole(w)
            whole(b)
        whole(st["out"][0])
        whole(st["out"][1])
        layout.append((start, len(st["wps"]), len(st["hidden"])))

    def chain(refs, xbfs, lay, final=None):
        start, npc, nh = lay
        k = 6 + start
        acc = None
        for j in range(npc):
            t = jnp.dot(xbfs[j], refs[k + j][...],
                        preferred_element_type=F32)
            acc = t if acc is None else acc + t
        k += npc
        h = jnp.maximum(acc + refs[k][...], 0.0)
        k += 1
        for _ in range(nh):
            h = jnp.maximum(
                jnp.dot(h.astype(BF16), refs[k][...],
                        preferred_element_type=F32) + refs[k + 1][...], 0.0)
            k += 2
        o = jnp.dot(h.astype(BF16), refs[k][...],
                    preferred_element_type=F32) + refs[k + 1][...]
        if final == "sigmoid":
            o = jax.nn.sigmoid(o)
        return o

    def body(*refs):
        out_ec1, out_ewn = refs[-2], refs[-1]
        g1a = refs[0][...].astype(BF16)
        g1b = refs[1][...].astype(BF16)
        e1v = refs[2][...].astype(BF16)
        eav = refs[3][...].astype(BF16)
        g0a = refs[4][...].astype(BF16)
        g0b = refs[5][...].astype(BF16)
        e2 = chain(refs, [g1a, g1b, e1v], layout[0]).astype(BF16)
        ew = chain(refs, [eav, e1v, e2], layout[1], final="sigmoid")
        ec1 = chain(refs, [g0a, g0b, ew.astype(BF16), eav, e1v, e2],
                    layout[2])
        out_ec1[...] = ec1
        out_ewn[...] = jnp.concatenate(
            [ew[:, DN * j:DN * j + 1] for j in range(G)], axis=1)

    return pl.pallas_call(
        body,
        grid=(grid,),
        in_specs=in_specs,
        out_specs=[pl.BlockSpec((BRP, 128), lambda i: (i, 0)),
                   pl.BlockSpec((BRP, G), lambda i: (i, 0))],
        out_shape=[jax.ShapeDtypeStruct((mp, 128), F32),
                   jax.ShapeDtypeStruct((mp, G), F32)],
    )(*operands)


def _node_mlp(n_out, br, pieces, b1, hidden, out_wt, out_b, final=None):
    grid = n_out // br
    in_specs = []
    operands = []
    for arr, blk_off, _ in pieces:
        in_specs.append(pl.BlockSpec((br, arr.shape[1]),
                                     lambda i, o=blk_off: (i + o, 0)))
        operands.append(arr)

    def whole(a):
        in_specs.append(pl.BlockSpec(a.shape, lambda i: (0, 0)))
        operands.append(a)

    for _, _, wp in pieces:
        whole(wp)
    whole(b1)
    for w, b in hidden:
        whole(w)
        whole(b)
    whole(out_wt)
    whole(out_b)

    n_p = len(pieces)
    n_h = len(hidden)
    out_width = out_wt.shape[1]

    def body(*refs):
        xs = refs[:n_p]
        wps = refs[n_p:2 * n_p]
        k = 2 * n_p
        b1r = refs[k]
        k += 1
        hs = []
        for _ in range(n_h):
            hs.append((refs[k], refs[k + 1]))
            k += 2
        owr, obr = refs[k], refs[k + 1]
        out_ref = refs[k + 2]
        acc = None
        for xr, wr in zip(xs, wps):
            t = jnp.dot(xr[...].astype(BF16), wr[...],
                        preferred_element_type=F32)
            acc = t if acc is None else acc + t
        h = jnp.maximum(acc + b1r[...], 0.0)
        for wr2, br2 in hs:
            h = jnp.maximum(
                jnp.dot(h.astype(BF16), wr2[...],
                        preferred_element_type=F32) + br2[...], 0.0)
        o = jnp.dot(h.astype(BF16), owr[...],
                    preferred_element_type=F32) + obr[...]
        if final == "sigmoid":
            o = jax.nn.sigmoid(o)
        out_ref[...] = o

    return pl.pallas_call(
        body,
        grid=(grid,),
        in_specs=in_specs,
        out_specs=pl.BlockSpec((br, out_width), lambda i: (i, 0)),
        out_shape=jax.ShapeDtypeStruct((n_out, out_width), F32),
    )(*operands)


def _repack_ea(ea, n_edges):
    mp = n_edges // G
    grid = mp // BRP
    gblk = mp // BRP

    def body(*refs):
        out_ref = refs[G]
        z = jnp.zeros((BRP, DN - 4), F32)
        out_ref[...] = jnp.concatenate(
            [jnp.concatenate([refs[g][...], z], axis=1) for g in range(G)],
            axis=1)

    return pl.pallas_call(
        body,
        grid=(grid,),
        in_specs=[pl.BlockSpec((BRP, 4), lambda i, g=g: (i + g * gblk, 0))
                  for g in range(G)],
        out_specs=pl.BlockSpec((BRP, 128), lambda i: (i, 0)),
        out_shape=jax.ShapeDtypeStruct((mp, 128), F32),
    )(*([ea] * G))


def _pad_rows(wt, rows):
    return jnp.pad(wt, ((0, rows - wt.shape[0]), (0, 0)))


def _pad_cols(wt, cols):
    return jnp.pad(wt, ((0, 0), (0, cols - wt.shape[1])))


def _bd(w):
    return jnp.kron(jnp.eye(G, dtype=w.dtype), w)


def kernel(x, edge_index, edge_attr, params):
    n_nodes = x.shape[0]
    n_edges = edge_index.shape[1]
    n_pad = ((n_nodes + NS * 8 - 1) // (NS * 8)) * (NS * 8)
    me = n_edges // G
    mg = 2 * n_edges // G
    OB = (n_edges // G) // BRP

    dst = edge_index[1].astype(jnp.int32)
    src = edge_index[0].astype(jnp.int32)
    idx_cat = jnp.concatenate([dst, src])
    idx_p = jnp.transpose(idx_cat.reshape(G, mg)).reshape(2 * n_edges)
    dst_p = jnp.transpose(dst.reshape(G, n_edges // G)).reshape(n_edges)

    xp = jnp.zeros((n_pad, DN), F32).at[:n_nodes, :3].set(x)
    eap = _repack_ea(edge_attr.astype(F32), n_edges)

    BR_N = n_pad // 16

    def bf(w):
        return w.astype(BF16)

    def rel_first_gather_w(W1t, h):
        Wd = _pad_rows(W1t[0:3], DN)
        Ws = _pad_rows(W1t[3:6], DN)
        WA = jnp.zeros((128, G * h), F32)
        WB = jnp.zeros((128, G * h), F32)
        for a in range(4):
            WA = WA.at[DN * a:DN * a + DN, 2 * a * h:(2 * a + 1) * h].set(Wd)
            WA = WA.at[64 + DN * a:64 + DN * a + DN,
                       2 * a * h:(2 * a + 1) * h].set(Ws)
            WB = WB.at[DN * a:DN * a + DN,
                       (2 * a + 1) * h:(2 * a + 2) * h].set(Wd)
            WB = WB.at[64 + DN * a:64 + DN * a + DN,
                       (2 * a + 1) * h:(2 * a + 2) * h].set(Ws)
        return WA, WB

    def rel_call(p, e_pieces, gpk):
        (W1, b1), (W2, b2), (W3, b3), (W4, b4) = p
        h = W1.shape[0]
        W1t = W1.T
        WA, WB = rel_first_gather_w(W1t, h)
        pieces = [(gpk, 0, bf(WA)), (gpk, OB, bf(WB))]
        col = 6
        for arr, d in e_pieces:
            wp = _pad_rows(W1t[col:col + d], DN)
            col += d
            pieces.append((arr, 0, bf(_bd(wp))))
        hidden = [(bf(_bd(W2.T)), jnp.tile(b2[None, :], (1, G))),
                  (bf(_bd(W3.T)), jnp.tile(b3[None, :], (1, G)))]
        owt = bf(_bd(_pad_cols(W4.T, DN)))
        ob = jnp.tile(_pad_cols(b4[None, :], DN), (1, G))
        return _edge_mlp(me, pieces, jnp.tile(b1[None, :], (1, G)),
                         hidden, owt, ob)

    def whead_call(p, e_pieces):
        (W1, b1), (W2, b2), (W3, b3), (W4, b4) = p
        h = W1.shape[0]
        W1t = W1.T
        pieces = []
        col = 0
        for arr, d in e_pieces:
            wp = _pad_rows(W1t[col:col + d], DN)
            col += d
            pieces.append((arr, 0, bf(_bd(wp))))
        hidden = [(bf(_bd(W2.T)), jnp.tile(b2[None, :], (1, G))),
                  (bf(_bd(W3.T)), jnp.tile(b3[None, :], (1, G)))]
        owt = bf(_bd(_pad_cols(W4.T, DN)))
        ob = jnp.tile(_pad_cols(b4[None, :], DN), (1, G))
        return _edge_mlp(me, pieces, jnp.tile(b1[None, :], (1, G)),
                         hidden, owt, ob, final="sigmoid", narrow=True)

    def obj_call(p, xtab, agg):
        (W1, b1), (W2, b2), (W3, b3), (W4, b4) = p
        W1t = W1.T
        eo = W1t.shape[0] - 3
        wx = bf(_pad_rows(W1t[0:3], DN))
        wa = bf(_pad_rows(W1t[3:3 + eo], DN))
        pieces = [(xtab, 0, wx), (agg, 0, wa), (agg, n_pad // BR_N, wa)]
        hidden = [(bf(W2.T), b2[None, :]), (bf(W3.T), b3[None, :])]
        return _node_mlp(n_pad, BR_N, pieces, b1[None, :], hidden,
                         bf(_pad_cols(W4.T, DN)), _pad_cols(b4[None, :], DN))

    def nhead_call(p, tabs, final):
        (W1, b1), (W2, b2), (W3, b3), (W4, b4) = p
        W1t = W1.T
        pieces = [(tab, 0, bf(_pad_rows(W1t[3 * i:3 * i + 3], DN)))
                  for i, tab in enumerate(tabs)]
        hidden = [(bf(W2.T), b2[None, :]), (bf(W3.T), b3[None, :])]
        return _node_mlp(n_pad, BR_N, pieces, b1[None, :], hidden,
                         bf(W4.T), b4[None, :], final=final)

    g0 = _gather_rows(xp, idx_p)
    gp0 = g0.reshape(mg, 128)
    e1 = rel_call(params['in_w1']['rel'], [(eap, 4)], gp0)
    a1 = _segment_sum(e1.reshape(n_edges, DN), dst_p, n_pad)
    x1 = obj_call(params['in_w1']['obj'], xp, a1)

    g1 = _gather_rows(x1, idx_p)
    gp1 = g1.reshape(mg, 128)

    def rel_wset(p, e_dims, with_gather=True):
        (W1, b1), (W2, b2), (W3, b3), (W4, b4) = p
        h = W1.shape[0]
        W1t = W1.T
        wps = []
        col = 0
        if with_gather:
            WA, WB = rel_first_gather_w(W1t, h)
            wps += [bf(WA), bf(WB)]
            col = 6
        for d in e_dims:
            wps.append(bf(_bd(_pad_rows(W1t[col:col + d], DN))))
            col += d
        return {
            "wps": wps,
            "b1": jnp.tile(b1[None, :], (1, G)),
            "hidden": [(bf(_bd(W2.T)), jnp.tile(b2[None, :], (1, G))),
                       (bf(_bd(W3.T)), jnp.tile(b3[None, :], (1, G)))],
            "out": (bf(_bd(_pad_cols(W4.T, DN))),
                    jnp.tile(_pad_cols(b4[None, :], DN), (1, G))),
        }

    sets = {
        "w2": rel_wset(params['in_w2']['rel'], [4]),
        "wh": rel_wset(params['W'], [4, 4, 4], with_gather=False),
        "c1": rel_wset(params['in_c1']['rel'], [1, 4, 4, 4]),
    }
    ec1, ewn = _fused_mid(me, gp0, gp1, e1, eap, sets)
    ac1 = _segment_sum(ec1.reshape(n_edges, DN), dst_p, n_pad)
    xc1 = obj_call(params['in_c1']['obj'], xp, ac1)

    gc2 = _gather_rows(xc1, idx_p)
    gpc2 = gc2.reshape(mg, 128)
    ec2 = rel_call(params['in_c2']['rel'], [(ec1, 8)], gpc2)
    ac2 = _segment_sum(ec2.reshape(n_edges, DN), dst_p, n_pad)
    xc2 = obj_call(params['in_c2']['obj'], xc1, ac2)

    gc3 = _gather_rows(xc2, idx_p)
    gpc3 = gc3.reshape(mg, 128)
    ec3 = rel_call(params['in_c3']['rel'], [(ec2, 8)], gpc3)
    ac3 = _segment_sum(ec3.reshape(n_edges, DN), dst_p, n_pad)
    xc3 = obj_call(params['in_c3']['obj'], xc2, ac3)

    tabs = [xp, xc1, xc2, xc3]
    beta = nhead_call(params['B'], tabs, "sigmoid")
    xc = nhead_call(params['X'], tabs, None)

    ew = jnp.transpose(ewn).reshape(n_edges, 1)
    return ew, xc[:n_nodes], beta[:n_nodes]
